# baseline (speedup 1.0000x reference)
import functools

import jax
import jax.numpy as jnp
from jax.experimental import pallas as pl
from jax.experimental.pallas import tpu as pltpu

_NOISE_SCALE = 0.01
_TOP_K_RATIO = 0.5
_ROWS = 256


_LN_C = (1.0000028668681713, -0.49992315126644116, 0.3327617641353723,
         -0.25364326648244867, 0.2181395213148837, -0.14166949261953013)
_SIN_C = (3.1415841334365413, -5.167241112502031, 2.5460347391019535,
          -0.5866649225180265, 0.06632052698300894)
_COS_C = (0.999999443211585, -4.934758576082029, 4.058163190630259,
          -1.3327490919641158, 0.2301265818255982, -0.02078393531124867)
_LN2 = 0.6931471805599453


def _fast_ln(x):
    ib = jax.lax.bitcast_convert_type(x, jnp.int32)
    e = (ib >> 23) - 127
    m = jax.lax.bitcast_convert_type((ib & 0x7FFFFF) | 0x3F800000,
                                     jnp.float32)
    big = m > 1.4142135623730951
    m = jnp.where(big, 0.5 * m, m)
    ef = (e + big.astype(jnp.int32)).astype(jnp.float32)
    t = m - 1.0
    p = jnp.float32(_LN_C[-1])
    for c in _LN_C[-2::-1]:
        p = p * t + c
    return ef * _LN2 + t * p


def _sincos_2pi(f):
    y = 2.0 * f - 1.0
    y2 = y * y
    ps = jnp.float32(_SIN_C[-1])
    for c in _SIN_C[-2::-1]:
        ps = ps * y2 + c
    pc = jnp.float32(_COS_C[-1])
    for c in _COS_C[-2::-1]:
        pc = pc * y2 + c
    return -y * ps, -pc


def _gaussian(shape):
    r_, d = shape
    half = (r_, d // 2)
    b1 = pltpu.prng_random_bits(half).astype(jnp.uint32)
    b2 = pltpu.prng_random_bits(half).astype(jnp.uint32)
    u1 = (b1 >> 8).astype(jnp.float32) * jnp.float32(2.0 ** -24) + \
        jnp.float32(2.0 ** -25)
    u2 = (b2 >> 8).astype(jnp.float32) * jnp.float32(2.0 ** -24)
    r = jnp.float32(_NOISE_SCALE) * jnp.sqrt(-2.0 * _fast_ln(u1))
    s, c = _sincos_2pi(u2)
    return jnp.concatenate([r * c, r * s], axis=1)


def _kth_largest_bits(mag_bits, k, g0):
    r, d = mag_bits.shape
    kf = jnp.float32(k)
    lo = jnp.zeros((r, 1), jnp.int32)
    c_lo = jnp.full((r, 1), float(d), jnp.float32)
    hi = jnp.full((r, 1), 0x7F800000, jnp.int32)
    c_hi = jnp.zeros((r, 1), jnp.float32)
    g = g0
    for _ in range(10):
        g = jnp.clip(g, lo + 1, jnp.maximum(hi - 1, lo + 1))
        cnt = jnp.sum((mag_bits >= g).astype(jnp.float32), axis=1,
                      keepdims=True)
        ge = cnt >= kf
        c_lo = jnp.where(ge, cnt, kf + (c_lo - kf) * 0.5)
        c_hi = jnp.where(ge, kf + (c_hi - kf) * 0.5, cnt)
        lo = jnp.where(ge, g, lo)
        hi = jnp.where(ge, hi, g)
        frac = (c_lo - kf) / jnp.maximum(c_lo - c_hi, jnp.float32(1e-3))
        g = lo + (frac * (hi - lo).astype(jnp.float32)).astype(jnp.int32)
    return lo


def _loss_kernel(s_ref, t_ref, out_ref, *, k, inv_2b, nblocks):
    i = pl.program_id(0)

    @pl.when(i == 0)
    def _init():
        out_ref[...] = jnp.zeros((1, 1), jnp.float32)

    s = s_ref[...]
    t = t_ref[...]

    ones_red = jnp.ones((s.shape[1], 128), jnp.float32)

    def _rowsum(x):
        return jax.lax.dot_general(
            x, ones_red, (((1,), (0,)), ((), ())),
            preferred_element_type=jnp.float32)[:, :1]

    dot1 = _rowsum(s * t)
    ns1 = _rowsum(s * s)
    nt1 = _rowsum(t * t)

    inv_d = jnp.float32(1.0 / s.shape[1])
    s_g0 = jax.lax.bitcast_convert_type(jnp.sqrt(ns1 * inv_d), jnp.int32)
    t_g0 = jax.lax.bitcast_convert_type(jnp.sqrt(nt1 * inv_d), jnp.int32)
    sbits = jax.lax.bitcast_convert_type(jnp.abs(s), jnp.int32)
    tbits = jax.lax.bitcast_convert_type(jnp.abs(t), jnp.int32)
    s_thr = _kth_largest_bits(sbits, k, s_g0)
    t_thr = _kth_largest_bits(tbits, k, t_g0)

    pltpu.prng_seed(12345, i)
    zs = _gaussian(s.shape)
    zt = _gaussian(t.shape)
    sp = s + jnp.where(sbits >= s_thr, zs, 0.0)
    tp = t + jnp.where(tbits >= t_thr, zt, 0.0)

    dot2 = _rowsum(sp * tp)
    ns2 = _rowsum(sp * sp)
    nt2 = _rowsum(tp * tp)

    eps = jnp.float32(1e-8)
    cos1 = dot1 / (jnp.maximum(jnp.sqrt(ns1), eps) *
                   jnp.maximum(jnp.sqrt(nt1), eps))
    cos2 = dot2 / (jnp.maximum(jnp.sqrt(ns2), eps) *
                   jnp.maximum(jnp.sqrt(nt2), eps))

    out_ref[...] += jnp.sum(cos1 + cos2).reshape(1, 1)

    @pl.when(i == nblocks - 1)
    def _fin():
        out_ref[...] = 1.0 - out_ref[...] * inv_2b


def kernel(student_features, teacher_features):
    b, d = student_features.shape
    k = int(d * _TOP_K_RATIO)
    rows = min(_ROWS, b)
    nblocks = b // rows

    body = functools.partial(_loss_kernel, k=k, inv_2b=1.0 / (2.0 * b),
                             nblocks=nblocks)
    out = pl.pallas_call(
        body,
        grid=(nblocks,),
        in_specs=[
            pl.BlockSpec((rows, d), lambda i: (i, 0)),
            pl.BlockSpec((rows, d), lambda i: (i, 0)),
        ],
        out_specs=pl.BlockSpec((1, 1), lambda i: (0, 0)),
        out_shape=jax.ShapeDtypeStruct((1, 1), jnp.float32),
        compiler_params=pltpu.CompilerParams(
            dimension_semantics=("arbitrary",)),
    )(student_features, teacher_features)
    return out.reshape(())


# speedup vs baseline: 1.0656x; 1.0656x over previous
import functools

import jax
import jax.numpy as jnp
from jax.experimental import pallas as pl
from jax.experimental.pallas import tpu as pltpu

_NOISE_SCALE = 0.01
_TOP_K_RATIO = 0.5
_ROWS = 256


_LN_C = (1.0000028668681713, -0.49992315126644116, 0.3327617641353723,
         -0.25364326648244867, 0.2181395213148837, -0.14166949261953013)
_SIN_C = (3.1415841334365413, -5.167241112502031, 2.5460347391019535,
          -0.5866649225180265, 0.06632052698300894)
_COS_C = (0.999999443211585, -4.934758576082029, 4.058163190630259,
          -1.3327490919641158, 0.2301265818255982, -0.02078393531124867)
_LN2 = 0.6931471805599453


def _fast_ln(x):
    ib = jax.lax.bitcast_convert_type(x, jnp.int32)
    e = (ib >> 23) - 127
    m = jax.lax.bitcast_convert_type((ib & 0x7FFFFF) | 0x3F800000,
                                     jnp.float32)
    big = m > 1.4142135623730951
    m = jnp.where(big, 0.5 * m, m)
    ef = (e + big.astype(jnp.int32)).astype(jnp.float32)
    t = m - 1.0
    p = jnp.float32(_LN_C[-1])
    for c in _LN_C[-2::-1]:
        p = p * t + c
    return ef * _LN2 + t * p


def _sincos_2pi(f):
    y = 2.0 * f - 1.0
    y2 = y * y
    ps = jnp.float32(_SIN_C[-1])
    for c in _SIN_C[-2::-1]:
        ps = ps * y2 + c
    pc = jnp.float32(_COS_C[-1])
    for c in _COS_C[-2::-1]:
        pc = pc * y2 + c
    return -y * ps, -pc


def _gaussian(shape):
    r_, d = shape
    half = (r_, d // 2)
    b1 = pltpu.prng_random_bits(half).astype(jnp.uint32)
    b2 = pltpu.prng_random_bits(half).astype(jnp.uint32)
    u1 = (b1 >> 8).astype(jnp.float32) * jnp.float32(2.0 ** -24) + \
        jnp.float32(2.0 ** -25)
    u2 = (b2 >> 8).astype(jnp.float32) * jnp.float32(2.0 ** -24)
    r = jnp.float32(_NOISE_SCALE) * jnp.sqrt(-2.0 * _fast_ln(u1))
    s, c = _sincos_2pi(u2)
    return jnp.concatenate([r * c, r * s], axis=1)


def _kth_largest_bits(mag_bits, k, g0):
    r, d = mag_bits.shape
    kf = jnp.float32(k)
    lo = jnp.zeros((r, 1), jnp.int32)
    c_lo = jnp.full((r, 1), float(d), jnp.float32)
    hi = jnp.full((r, 1), 0x7F800000, jnp.int32)
    c_hi = jnp.zeros((r, 1), jnp.float32)
    g = g0
    for _ in range(10):
        g = jnp.clip(g, lo + 1, jnp.maximum(hi - 1, lo + 1))
        cnt = jnp.sum((mag_bits >= g).astype(jnp.float32), axis=1,
                      keepdims=True)
        ge = cnt >= kf
        c_lo = jnp.where(ge, cnt, kf + (c_lo - kf) * 0.5)
        c_hi = jnp.where(ge, kf + (c_hi - kf) * 0.5, cnt)
        lo = jnp.where(ge, g, lo)
        hi = jnp.where(ge, hi, g)
        frac = (c_lo - kf) / jnp.maximum(c_lo - c_hi, jnp.float32(1e-3))
        g = lo + (frac * (hi - lo).astype(jnp.float32)).astype(jnp.int32)
    return lo


def _loss_kernel(s_ref, t_ref, out_ref, *, k, inv_2b, nblocks):
    i = pl.program_id(0)

    @pl.when(i == 0)
    def _init():
        out_ref[...] = jnp.zeros((1, 1), jnp.float32)

    s = s_ref[...]
    t = t_ref[...]

    dot1 = jnp.sum(s * t, axis=1, keepdims=True)
    ns1 = jnp.sum(s * s, axis=1, keepdims=True)
    nt1 = jnp.sum(t * t, axis=1, keepdims=True)

    inv_d = jnp.float32(1.0 / s.shape[1])
    s_g0 = jax.lax.bitcast_convert_type(jnp.sqrt(ns1 * inv_d), jnp.int32)
    t_g0 = jax.lax.bitcast_convert_type(jnp.sqrt(nt1 * inv_d), jnp.int32)
    sbits = jax.lax.bitcast_convert_type(jnp.abs(s), jnp.int32)
    tbits = jax.lax.bitcast_convert_type(jnp.abs(t), jnp.int32)
    s_thr = _kth_largest_bits(sbits, k, s_g0)
    t_thr = _kth_largest_bits(tbits, k, t_g0)

    pltpu.prng_seed(12345, i)
    zs = _gaussian(s.shape)
    zt = _gaussian(t.shape)
    sp = s + jnp.where(sbits >= s_thr, zs, 0.0)
    tp = t + jnp.where(tbits >= t_thr, zt, 0.0)

    dot2 = jnp.sum(sp * tp, axis=1, keepdims=True)
    ns2 = jnp.sum(sp * sp, axis=1, keepdims=True)
    nt2 = jnp.sum(tp * tp, axis=1, keepdims=True)

    eps = jnp.float32(1e-8)
    cos1 = dot1 / (jnp.maximum(jnp.sqrt(ns1), eps) *
                   jnp.maximum(jnp.sqrt(nt1), eps))
    cos2 = dot2 / (jnp.maximum(jnp.sqrt(ns2), eps) *
                   jnp.maximum(jnp.sqrt(nt2), eps))

    out_ref[...] += jnp.sum(cos1 + cos2).reshape(1, 1)

    @pl.when(i == nblocks - 1)
    def _fin():
        out_ref[...] = 1.0 - out_ref[...] * inv_2b


def kernel(student_features, teacher_features):
    b, d = student_features.shape
    k = int(d * _TOP_K_RATIO)
    rows = min(_ROWS, b)
    nblocks = b // rows

    body = functools.partial(_loss_kernel, k=k, inv_2b=1.0 / (2.0 * b),
                             nblocks=nblocks)
    out = pl.pallas_call(
        body,
        grid=(nblocks,),
        in_specs=[
            pl.BlockSpec((rows, d), lambda i: (i, 0)),
            pl.BlockSpec((rows, d), lambda i: (i, 0)),
        ],
        out_specs=pl.BlockSpec((1, 1), lambda i: (0, 0)),
        out_shape=jax.ShapeDtypeStruct((1, 1), jnp.float32),
        compiler_params=pltpu.CompilerParams(
            dimension_semantics=("arbitrary",)),
    )(student_features, teacher_features)
    return out.reshape(())
